# baseline (speedup 1.0000x reference)
import functools

import jax
import jax.numpy as jnp
import numpy as np
from jax import lax
from jax.experimental import pallas as pl
from jax.experimental.pallas import tpu as pltpu
from jax.experimental.pallas import tpu_sc as plsc

_RADIUS = 0.4
_MAXG = 96.0
_MAXC = 128
_B = 2
_N = 4096
_NW = 32
_WPB = _NW // _B
_OWN = _B * _N // _NW
_NCHUNK = 8
_ROWS = _N // _WPB // _NCHUNK
_NGRP = _N // 16

_SIGN = np.int32(-2147483648)
_R2 = np.float32(_RADIUS * _RADIUS)

_PACK = np.zeros((_N, _NGRP), np.float32)
_PACK[np.arange(_N), np.arange(_N) >> 4] = (2.0 ** (np.arange(_N) & 15))
_PACK2 = np.zeros((_NGRP, 16), np.float32)
_PACK2[np.arange(_NGRP), np.arange(_NGRP) >> 4] = (2.0 ** (np.arange(_NGRP) & 15))


def _cells_body(x_ref, y_ref, z_ref, key_ref):
    r32 = jnp.float32(_RADIUS)
    for b in range(_B):
        arrs = (x_ref[b], y_ref[b], z_ref[b])
        cells = []
        gds = []
        for arr in arrs:
            lo = jnp.min(arr)
            up = jnp.max(arr)
            gdim = jnp.ceil(jnp.clip((up - lo) / r32, 0.0, jnp.float32(_MAXG)))
            center = (lo + up) / 2.0
            low2 = center - gdim * r32 / 2.0
            cell = jnp.floor((arr - low2) / r32)
            cell = jnp.clip(cell, 0.0, gdim - 1.0)
            cells.append(cell.astype(jnp.int32))
            gds.append(gdim.astype(jnp.int32))
        cid = (cells[0] * gds[1] + cells[1]) * gds[2] + cells[2]
        pos = (lax.broadcasted_iota(jnp.int32, arrs[0].shape, 0) * arrs[0].shape[1]
               + lax.broadcasted_iota(jnp.int32, arrs[0].shape, 1))
        key_ref[b] = cid * 4096 + pos


def _compute_keys(xs, ys, zs):
    shape = (_B, _N // 128, 128)
    key = pl.pallas_call(
        _cells_body,
        out_shape=jax.ShapeDtypeStruct(shape, jnp.int32),
    )(xs.reshape(shape), ys.reshape(shape), zs.reshape(shape))
    return key.reshape(_B * _N)


def _rank_body(a_ref, b_ref, ones_ref, rank_ref):
    ki = a_ref[...] ^ _SIGN
    kj = b_ref[0] ^ _SIGN
    lt = jnp.where(kj < ki, 1.0, 0.0)
    rank = lax.dot_general(lt, ones_ref[...], (((1,), (0,)), ((), ())),
                           precision=lax.Precision.DEFAULT)
    rank_ref[...] = rank.astype(jnp.int32)


def _compute_rank(key):
    key_col = key.reshape(_B * _N, 1)
    key_row = key.reshape(_B, 1, _N)
    rank = pl.pallas_call(
        _rank_body,
        grid=(_B, _N // 128),
        in_specs=[
            pl.BlockSpec((128, 1), lambda b, r: (b * (_N // 128) + r, 0)),
            pl.BlockSpec((1, 1, _N), lambda b, r: (b, 0, 0)),
            pl.BlockSpec((_N, 1), lambda b, r: (0, 0)),
        ],
        out_specs=pl.BlockSpec((128, 1), lambda b, r: (b * (_N // 128) + r, 0)),
        out_shape=jax.ShapeDtypeStruct((_B * _N, 1), jnp.int32),
    )(key_col, key_row, jnp.ones((_N, 1), jnp.float32))
    return rank.reshape(_B * _N)


@functools.cache
def _make_scatter():
    mesh = plsc.VectorSubcoreMesh(core_axis_name="c", subcore_axis_name="s")
    return functools.partial(
        pl.kernel,
        out_type=[
            jax.ShapeDtypeStruct((_B * _N,), jnp.int32),
            jax.ShapeDtypeStruct((_B * _N,), jnp.float32),
            jax.ShapeDtypeStruct((_B * _N,), jnp.float32),
            jax.ShapeDtypeStruct((_B * _N,), jnp.float32),
        ],
        mesh=mesh,
        compiler_params=pltpu.CompilerParams(needs_layout_passes=False),
        scratch_types=[
            pltpu.VMEM((_OWN,), jnp.int32),
            pltpu.VMEM((_OWN,), jnp.int32),
            pltpu.VMEM((_OWN,), jnp.float32),
            pltpu.VMEM((_OWN,), jnp.float32),
            pltpu.VMEM((_OWN,), jnp.float32),
            pltpu.VMEM((2, 128), jnp.int32),
            pltpu.SemaphoreType.DMA,
        ],
    )(_scatter_body)


def _scatter_body(key_hbm, rank_hbm, x_hbm, y_hbm, z_hbm,
                  key_s_hbm, xs_hbm, ys_hbm, zs_hbm,
                  ko_v, ro_v, xo_v, yo_v, zo_v, idx_v, sem):
    wid = lax.axis_index("s") * 2 + lax.axis_index("c")
    b = wid // _WPB
    bbase = b * _N
    obase = (wid % _WPB) * _OWN

    own = pl.ds(bbase + obase, _OWN)
    ins = [pltpu.async_copy(key_hbm.at[own], ko_v, sem),
           pltpu.async_copy(rank_hbm.at[own], ro_v, sem),
           pltpu.async_copy(x_hbm.at[own], xo_v, sem),
           pltpu.async_copy(y_hbm.at[own], yo_v, sem),
           pltpu.async_copy(z_hbm.at[own], zo_v, sem)]
    for cp in ins:
        cp.wait()

    def mkidx(c, _):
        idx_v[c // 8, pl.ds((c % 8) * 16, 16)] = \
            ro_v[pl.ds(c * 16, 16)] + bbase
        return 0

    lax.fori_loop(0, _OWN // 16, mkidx, 0)

    cps = []
    for c in range(2):
        sl = pl.ds(c * 128, 128)
        cps.append(pltpu.async_copy(ko_v.at[sl], key_s_hbm.at[idx_v.at[c]], sem))
        cps.append(pltpu.async_copy(xo_v.at[sl], xs_hbm.at[idx_v.at[c]], sem))
        cps.append(pltpu.async_copy(yo_v.at[sl], ys_hbm.at[idx_v.at[c]], sem))
        cps.append(pltpu.async_copy(zo_v.at[sl], zs_hbm.at[idx_v.at[c]], sem))
    for cp in cps:
        cp.wait()


def _mask_body(a_ref, bT_ref, p_ref, p2_ref, bits_ref, l2_ref):
    a = a_ref[0]
    bT = bT_ref[0]
    x_i, y_i, z_i = a[:, 0:1], a[:, 1:2], a[:, 2:3]
    sq_i = (x_i * x_i + y_i * y_i) + z_i * z_i
    bx, by, bz = bT[0:1, :], bT[1:2, :], bT[2:3, :]
    sq_j = (bx * bx + by * by) + bz * bz
    dot = lax.dot_general(a, bT, (((1,), (0,)), ((), ())),
                          precision=lax.Precision.DEFAULT)
    d2 = sq_i + sq_j - 2.0 * dot
    maskf = jnp.where(d2 <= _R2, 1.0, 0.0)
    pack = lax.dot_general(maskf, p_ref[...], (((1,), (0,)), ((), ())),
                           precision=lax.Precision.DEFAULT)
    bits_ref[0] = pack.astype(jnp.int32)
    nzw = jnp.where(pack != 0.0, 1.0, 0.0)
    l2 = lax.dot_general(nzw, p2_ref[...], (((1,), (0,)), ((), ())),
                         precision=lax.Precision.DEFAULT)
    l2_ref[...] = l2.astype(jnp.int32)


def _compute_bits(locs_s, locs_sT):
    return pl.pallas_call(
        _mask_body,
        grid=(_B, _N // 128),
        in_specs=[
            pl.BlockSpec((1, 128, 3), lambda b, r: (b, r, 0)),
            pl.BlockSpec((1, 3, _N), lambda b, r: (b, 0, 0)),
            pl.BlockSpec((_N, _NGRP), lambda b, r: (0, 0)),
            pl.BlockSpec((_NGRP, 16), lambda b, r: (0, 0)),
        ],
        out_specs=[
            pl.BlockSpec((1, 128, _NGRP), lambda b, r: (b, r, 0)),
            pl.BlockSpec((128, 16), lambda b, r: (b * (_N // 128) + r, 0)),
        ],
        out_shape=[
            jax.ShapeDtypeStruct((_B, _N, _NGRP), jnp.int32),
            jax.ShapeDtypeStruct((_B * _N, 16), jnp.int32),
        ],
    )(locs_s, locs_sT, jnp.asarray(_PACK), jnp.asarray(_PACK2))


@functools.cache
def _make_emit():
    mesh = plsc.VectorSubcoreMesh(core_axis_name="c", subcore_axis_name="s")
    return functools.partial(
        pl.kernel,
        out_type=[
            jax.ShapeDtypeStruct((_B * _N,), jnp.float32),
            jax.ShapeDtypeStruct((_B * _N * _MAXC,), jnp.float32),
        ],
        mesh=mesh,
        compiler_params=pltpu.CompilerParams(needs_layout_passes=False),
        scratch_types=[
            pltpu.VMEM((_N,), jnp.int32),
            pltpu.VMEM((_N,), jnp.float32),
            pltpu.VMEM((_ROWS, _NGRP), jnp.int32),
            pltpu.VMEM((_ROWS, 16), jnp.int32),
            pltpu.VMEM((_ROWS * 144,), jnp.float32),
            pltpu.SemaphoreType.DMA,
        ],
    )(_emit_body)


def _emit_body(key_s_hbm, bits_hbm, l2_hbm, idxs_hbm, nbr_hbm,
               keys_v, ordf_v, rows_v, l2rows_v, out_v, sem):
    wid = lax.axis_index("s") * 2 + lax.axis_index("c")
    b = wid // _WPB
    ws = wid % _WPB
    bbase = b * _N

    pltpu.sync_copy(key_s_hbm.at[pl.ds(bbase, _N)], keys_v)

    iota = lax.iota(jnp.int32, 16)

    def unpack(c, _):
        sl = pl.ds(c * 16, 16)
        ordf_v[sl] = (keys_v[sl] & 4095).astype(jnp.float32)
        return 0

    lax.fori_loop(0, _N // 16, unpack, 0)

    def per_chunk(c, _):
        row0 = (c * _WPB + ws) * _ROWS
        cp = pltpu.async_copy(bits_hbm.at[pl.ds(bbase + row0, _ROWS)],
                              rows_v, sem)
        cp2 = pltpu.async_copy(l2_hbm.at[pl.ds(bbase + row0, _ROWS)],
                               l2rows_v, sem)

        def fill(k, _):
            out_v[pl.ds(k * 16, 16)] = jnp.full((16,), -1.0, jnp.float32)
            return 0

        lax.fori_loop(0, _ROWS * 144 // 16, fill, 0)
        cp.wait()
        cp2.wait()

        def per_row(p, _):
            bp16 = jnp.broadcast_to(p, (16,))
            l2 = l2rows_v[p, pl.ds(0, 16)]
            nz2 = l2 != 0
            n2 = plsc.all_reduce_population_count(nz2)[0]

            def cond2(carry):
                nz2, cnt, n2 = carry
                return (n2 > 0) & (cnt < _MAXC)

            def body2(carry):
                nz2, cnt, n2 = carry
                g = plsc.all_reduce_ffs(nz2)
                gbits = plsc.load_gather(l2rows_v, [bp16, g])
                nzw = ((lax.shift_right_logical(gbits, iota)) & 1) == 1
                nw = plsc.all_reduce_population_count(nzw)[0]

                def cond(carry):
                    nzw, cnt, nw = carry
                    return (nw > 0) & (cnt < _MAXC)

                def body(carry):
                    nzw, cnt, nw = carry
                    l = plsc.all_reduce_ffs(nzw)
                    w = plsc.load_gather(rows_v, [bp16, g * 16 + l])
                    hits = ((lax.shift_right_logical(w, iota)) & 1) == 1
                    jv = ((g * 16 + l) * 16 + iota).astype(jnp.float32)
                    plsc.store_compressed(
                        out_v.at[pl.ds(p * 144 + cnt, 16)], jv, mask=hits)
                    nhit = plsc.all_reduce_population_count(hits)[0]
                    return (nzw & (iota != l), cnt + nhit, nw - 1)

                _, cnt, _ = lax.while_loop(cond, body, (nzw, cnt, nw))
                return (nz2 & (iota != g), cnt, n2 - 1)

            lax.while_loop(cond2, body2, (nz2, jnp.int32(0), n2))
            return 0

        lax.fori_loop(0, _ROWS, per_row, 0)

        pltpu.sync_copy(ordf_v.at[pl.ds(row0, _ROWS)],
                        idxs_hbm.at[pl.ds(bbase + row0, _ROWS)])
        cps = [pltpu.async_copy(
                   out_v.at[pl.ds(r * 144, _MAXC)],
                   nbr_hbm.at[pl.ds((bbase + row0 + r) * _MAXC, _MAXC)],
                   sem)
               for r in range(_ROWS)]
        for c2 in cps:
            c2.wait()
        return 0

    lax.fori_loop(0, _NCHUNK, per_chunk, 0)


@jax.jit
def kernel(locs):
    xs = locs[..., 0].reshape(_B * _N)
    ys = locs[..., 1].reshape(_B * _N)
    zs = locs[..., 2].reshape(_B * _N)
    key = _compute_keys(xs, ys, zs)
    rank = _compute_rank(key)
    key_s, xs_s, ys_s, zs_s = _make_scatter()(key, rank, xs, ys, zs)
    locs_s = jnp.stack(
        [xs_s.reshape(_B, _N), ys_s.reshape(_B, _N), zs_s.reshape(_B, _N)],
        axis=-1)
    locs_sT = jnp.stack(
        [xs_s.reshape(_B, _N), ys_s.reshape(_B, _N), zs_s.reshape(_B, _N)],
        axis=1)
    bits, l2 = _compute_bits(locs_s, locs_sT)
    idxs, nbrs = _make_emit()(key_s, bits.reshape(_B * _N, _NGRP), l2)
    return idxs.reshape(_B, _N), nbrs.reshape(_B, _N, _MAXC)


# speedup vs baseline: 1.0017x; 1.0017x over previous
import functools

import jax
import jax.numpy as jnp
import numpy as np
from jax import lax
from jax.experimental import pallas as pl
from jax.experimental.pallas import tpu as pltpu
from jax.experimental.pallas import tpu_sc as plsc

_RADIUS = 0.4
_MAXG = 96.0
_MAXC = 128
_B = 2
_N = 4096
_NW = 32
_WPB = _NW // _B
_OWN = _B * _N // _NW
_NCHUNK = 8
_ROWS = _N // _WPB // _NCHUNK
_NGRP = _N // 16

_SIGN = np.int32(-2147483648)
_R2 = np.float32(_RADIUS * _RADIUS)

_PACK = np.zeros((_N, _NGRP), np.float32)
_PACK[np.arange(_N), np.arange(_N) >> 4] = (2.0 ** (np.arange(_N) & 15))
_PACK2 = np.zeros((_NGRP, 16), np.float32)
_PACK2[np.arange(_NGRP), np.arange(_NGRP) >> 4] = (2.0 ** (np.arange(_NGRP) & 15))


def _cells_body(x_ref, y_ref, z_ref, key_ref):
    r32 = jnp.float32(_RADIUS)
    for b in range(_B):
        arrs = (x_ref[b], y_ref[b], z_ref[b])
        cells = []
        gds = []
        for arr in arrs:
            lo = jnp.min(arr)
            up = jnp.max(arr)
            gdim = jnp.ceil(jnp.clip((up - lo) / r32, 0.0, jnp.float32(_MAXG)))
            center = (lo + up) / 2.0
            low2 = center - gdim * r32 / 2.0
            cell = jnp.floor((arr - low2) / r32)
            cell = jnp.clip(cell, 0.0, gdim - 1.0)
            cells.append(cell.astype(jnp.int32))
            gds.append(gdim.astype(jnp.int32))
        cid = (cells[0] * gds[1] + cells[1]) * gds[2] + cells[2]
        pos = (lax.broadcasted_iota(jnp.int32, arrs[0].shape, 0) * arrs[0].shape[1]
               + lax.broadcasted_iota(jnp.int32, arrs[0].shape, 1))
        key_ref[b] = cid * 4096 + pos


def _compute_keys(xs, ys, zs):
    shape = (_B, _N // 128, 128)
    key = pl.pallas_call(
        _cells_body,
        out_shape=jax.ShapeDtypeStruct(shape, jnp.int32),
    )(xs.reshape(shape), ys.reshape(shape), zs.reshape(shape))
    return key.reshape(_B * _N)


def _rank_body(a_ref, b_ref, ones_ref, rank_ref):
    ki = a_ref[...] ^ _SIGN
    kj = b_ref[0] ^ _SIGN
    lt = jnp.where(kj < ki, 1.0, 0.0)
    rank = lax.dot_general(lt, ones_ref[...], (((1,), (0,)), ((), ())),
                           precision=lax.Precision.DEFAULT)
    rank_ref[...] = rank.astype(jnp.int32)


def _compute_rank(key):
    key_col = key.reshape(_B * _N, 1)
    key_row = key.reshape(_B, 1, _N)
    rank = pl.pallas_call(
        _rank_body,
        grid=(_B, _N // 128),
        in_specs=[
            pl.BlockSpec((128, 1), lambda b, r: (b * (_N // 128) + r, 0)),
            pl.BlockSpec((1, 1, _N), lambda b, r: (b, 0, 0)),
            pl.BlockSpec((_N, 1), lambda b, r: (0, 0)),
        ],
        out_specs=pl.BlockSpec((128, 1), lambda b, r: (b * (_N // 128) + r, 0)),
        out_shape=jax.ShapeDtypeStruct((_B * _N, 1), jnp.int32),
    )(key_col, key_row, jnp.ones((_N, 1), jnp.float32))
    return rank.reshape(_B * _N)


@functools.cache
def _make_scatter():
    mesh = plsc.VectorSubcoreMesh(core_axis_name="c", subcore_axis_name="s")
    return functools.partial(
        pl.kernel,
        out_type=[
            jax.ShapeDtypeStruct((_B * _N,), jnp.int32),
            jax.ShapeDtypeStruct((_B * _N,), jnp.float32),
            jax.ShapeDtypeStruct((_B * _N,), jnp.float32),
            jax.ShapeDtypeStruct((_B * _N,), jnp.float32),
        ],
        mesh=mesh,
        compiler_params=pltpu.CompilerParams(needs_layout_passes=False),
        scratch_types=[
            pltpu.VMEM((_OWN,), jnp.int32),
            pltpu.VMEM((_OWN,), jnp.int32),
            pltpu.VMEM((_OWN,), jnp.float32),
            pltpu.VMEM((_OWN,), jnp.float32),
            pltpu.VMEM((_OWN,), jnp.float32),
            pltpu.VMEM((2, 128), jnp.int32),
            pltpu.SemaphoreType.DMA,
        ],
    )(_scatter_body)


def _scatter_body(key_hbm, rank_hbm, x_hbm, y_hbm, z_hbm,
                  key_s_hbm, xs_hbm, ys_hbm, zs_hbm,
                  ko_v, ro_v, xo_v, yo_v, zo_v, idx_v, sem):
    wid = lax.axis_index("s") * 2 + lax.axis_index("c")
    b = wid // _WPB
    bbase = b * _N
    obase = (wid % _WPB) * _OWN

    own = pl.ds(bbase + obase, _OWN)
    ins = [pltpu.async_copy(key_hbm.at[own], ko_v, sem),
           pltpu.async_copy(rank_hbm.at[own], ro_v, sem),
           pltpu.async_copy(x_hbm.at[own], xo_v, sem),
           pltpu.async_copy(y_hbm.at[own], yo_v, sem),
           pltpu.async_copy(z_hbm.at[own], zo_v, sem)]
    for cp in ins:
        cp.wait()

    def mkidx(c, _):
        idx_v[c // 8, pl.ds((c % 8) * 16, 16)] = \
            ro_v[pl.ds(c * 16, 16)] + bbase
        return 0

    lax.fori_loop(0, _OWN // 16, mkidx, 0)

    cps = []
    for c in range(2):
        sl = pl.ds(c * 128, 128)
        cps.append(pltpu.async_copy(ko_v.at[sl], key_s_hbm.at[idx_v.at[c]], sem))
        cps.append(pltpu.async_copy(xo_v.at[sl], xs_hbm.at[idx_v.at[c]], sem))
        cps.append(pltpu.async_copy(yo_v.at[sl], ys_hbm.at[idx_v.at[c]], sem))
        cps.append(pltpu.async_copy(zo_v.at[sl], zs_hbm.at[idx_v.at[c]], sem))
    for cp in cps:
        cp.wait()


def _mask_body(a_ref, bT_ref, p_ref, p2_ref, bits_ref, l2_ref):
    a = a_ref[0]
    bT = bT_ref[0]
    x_i, y_i, z_i = a[:, 0:1], a[:, 1:2], a[:, 2:3]
    sq_i = (x_i * x_i + y_i * y_i) + z_i * z_i
    bx, by, bz = bT[0:1, :], bT[1:2, :], bT[2:3, :]
    sq_j = (bx * bx + by * by) + bz * bz
    dot = lax.dot_general(a, bT, (((1,), (0,)), ((), ())),
                          precision=lax.Precision.DEFAULT)
    d2 = sq_i + sq_j - 2.0 * dot
    maskf = jnp.where(d2 <= _R2, 1.0, 0.0)
    pack = lax.dot_general(maskf, p_ref[...], (((1,), (0,)), ((), ())),
                           precision=lax.Precision.DEFAULT)
    bits_ref[0] = pack.astype(jnp.int32)
    nzw = jnp.where(pack != 0.0, 1.0, 0.0)
    l2 = lax.dot_general(nzw, p2_ref[...], (((1,), (0,)), ((), ())),
                         precision=lax.Precision.DEFAULT)
    l2_ref[...] = l2.astype(jnp.int32)


def _compute_bits(locs_s, locs_sT):
    return pl.pallas_call(
        _mask_body,
        grid=(_B, _N // 128),
        in_specs=[
            pl.BlockSpec((1, 128, 3), lambda b, r: (b, r, 0)),
            pl.BlockSpec((1, 3, _N), lambda b, r: (b, 0, 0)),
            pl.BlockSpec((_N, _NGRP), lambda b, r: (0, 0)),
            pl.BlockSpec((_NGRP, 16), lambda b, r: (0, 0)),
        ],
        out_specs=[
            pl.BlockSpec((1, 128, _NGRP), lambda b, r: (b, r, 0)),
            pl.BlockSpec((128, 16), lambda b, r: (b * (_N // 128) + r, 0)),
        ],
        out_shape=[
            jax.ShapeDtypeStruct((_B, _N, _NGRP), jnp.int32),
            jax.ShapeDtypeStruct((_B * _N, 16), jnp.int32),
        ],
    )(locs_s, locs_sT, jnp.asarray(_PACK), jnp.asarray(_PACK2))


@functools.cache
def _make_emit():
    mesh = plsc.VectorSubcoreMesh(core_axis_name="c", subcore_axis_name="s")
    return functools.partial(
        pl.kernel,
        out_type=[
            jax.ShapeDtypeStruct((_B * _N,), jnp.float32),
            jax.ShapeDtypeStruct((_B * _N * _MAXC,), jnp.float32),
        ],
        mesh=mesh,
        compiler_params=pltpu.CompilerParams(needs_layout_passes=False),
        scratch_types=[
            pltpu.VMEM((_N,), jnp.int32),
            pltpu.VMEM((_N,), jnp.float32),
            pltpu.VMEM((_ROWS, _NGRP), jnp.int32),
            pltpu.VMEM((_ROWS, 16), jnp.int32),
            pltpu.VMEM((_ROWS * 144,), jnp.float32),
            pltpu.SemaphoreType.DMA,
        ],
    )(_emit_body)


def _emit_body(key_s_hbm, bits_hbm, l2_hbm, idxs_hbm, nbr_hbm,
               keys_v, ordf_v, rows_v, l2rows_v, out_v, sem):
    wid = lax.axis_index("s") * 2 + lax.axis_index("c")
    b = wid // _WPB
    ws = wid % _WPB
    bbase = b * _N

    pltpu.sync_copy(key_s_hbm.at[pl.ds(bbase, _N)], keys_v)

    iota = lax.iota(jnp.int32, 16)

    def unpack(c, _):
        sl = pl.ds(c * 16, 16)
        ordf_v[sl] = (keys_v[sl] & 4095).astype(jnp.float32)
        return 0

    lax.fori_loop(0, _N // 16, unpack, 0)

    def per_chunk(c, _):
        row0 = (c * _WPB + ws) * _ROWS
        cp = pltpu.async_copy(bits_hbm.at[pl.ds(bbase + row0, _ROWS)],
                              rows_v, sem)
        cp2 = pltpu.async_copy(l2_hbm.at[pl.ds(bbase + row0, _ROWS)],
                               l2rows_v, sem)

        def fill(k, _):
            out_v[pl.ds(k * 16, 16)] = jnp.full((16,), -1.0, jnp.float32)
            return 0

        lax.fori_loop(0, _ROWS * 144 // 16, fill, 0)
        cp.wait()
        cp2.wait()

        @plsc.parallel_loop(0, _ROWS, unroll=2)
        def per_row(p):
            bp16 = jnp.broadcast_to(p, (16,))
            l2 = l2rows_v[p, pl.ds(0, 16)]
            nz2 = l2 != 0
            n2 = plsc.all_reduce_population_count(nz2)[0]

            def cond2(carry):
                nz2, cnt, n2 = carry
                return (n2 > 0) & (cnt < _MAXC)

            def body2(carry):
                nz2, cnt, n2 = carry
                g = plsc.all_reduce_ffs(nz2)
                gbits = plsc.load_gather(l2rows_v, [bp16, g])
                nzw = ((lax.shift_right_logical(gbits, iota)) & 1) == 1
                nw = plsc.all_reduce_population_count(nzw)[0]

                def cond(carry):
                    nzw, cnt, nw = carry
                    return (nw > 0) & (cnt < _MAXC)

                def body(carry):
                    nzw, cnt, nw = carry
                    l = plsc.all_reduce_ffs(nzw)
                    w = plsc.load_gather(rows_v, [bp16, g * 16 + l])
                    hits = ((lax.shift_right_logical(w, iota)) & 1) == 1
                    jv = ((g * 16 + l) * 16 + iota).astype(jnp.float32)
                    plsc.store_compressed(
                        out_v.at[pl.ds(p * 144 + cnt, 16)], jv, mask=hits)
                    nhit = plsc.all_reduce_population_count(hits)[0]
                    return (nzw & (iota != l), cnt + nhit, nw - 1)

                _, cnt, _ = lax.while_loop(cond, body, (nzw, cnt, nw))
                return (nz2 & (iota != g), cnt, n2 - 1)

            lax.while_loop(cond2, body2, (nz2, jnp.int32(0), n2))

        pltpu.sync_copy(ordf_v.at[pl.ds(row0, _ROWS)],
                        idxs_hbm.at[pl.ds(bbase + row0, _ROWS)])
        cps = [pltpu.async_copy(
                   out_v.at[pl.ds(r * 144, _MAXC)],
                   nbr_hbm.at[pl.ds((bbase + row0 + r) * _MAXC, _MAXC)],
                   sem)
               for r in range(_ROWS)]
        for c2 in cps:
            c2.wait()
        return 0

    lax.fori_loop(0, _NCHUNK, per_chunk, 0)


@jax.jit
def kernel(locs):
    xs = locs[..., 0].reshape(_B * _N)
    ys = locs[..., 1].reshape(_B * _N)
    zs = locs[..., 2].reshape(_B * _N)
    key = _compute_keys(xs, ys, zs)
    rank = _compute_rank(key)
    key_s, xs_s, ys_s, zs_s = _make_scatter()(key, rank, xs, ys, zs)
    locs_s = jnp.stack(
        [xs_s.reshape(_B, _N), ys_s.reshape(_B, _N), zs_s.reshape(_B, _N)],
        axis=-1)
    locs_sT = jnp.stack(
        [xs_s.reshape(_B, _N), ys_s.reshape(_B, _N), zs_s.reshape(_B, _N)],
        axis=1)
    bits, l2 = _compute_bits(locs_s, locs_sT)
    idxs, nbrs = _make_emit()(key_s, bits.reshape(_B * _N, _NGRP), l2)
    return idxs.reshape(_B, _N), nbrs.reshape(_B, _N, _MAXC)
